# Initial kernel scaffold; baseline (speedup 1.0000x reference)
import jax
import jax.numpy as jnp
from jax.experimental import pallas as pl


def kernel(x, edge_index, edge_attr, pos, embed_W, embed_b, edge_W1, edge_b1, edge_W2, edge_b2, coors_W1, coors_b1, coors_W2, coors_b2, node_W1, node_b1, node_W2, node_b2, lin_W, lin_b):
    raise NotImplementedError("write your pallas kernel here")


# speedup vs baseline: 3.3307x; 3.3307x over previous
import functools

import jax
import jax.numpy as jnp
from jax import lax
from jax.experimental import pallas as pl
from jax.experimental.pallas import tpu as pltpu
from jax.experimental.pallas import tpu_sc as plsc

NC = 2
NS = 16
NW = NC * NS
CH = 128

WD = 32
H = 16


def _silu(v):
    return v * jax.nn.sigmoid(v)


@functools.lru_cache(maxsize=None)
def _make_gather(N, E):
    per_w = E // NW
    n_full = per_w // CH
    tail = per_w - n_full * CH
    mesh = plsc.VectorSubcoreMesh(core_axis_name="c", subcore_axis_name="s")

    def body(table_h, src_h, dst_h, srows_h, drows_h, idx_v, rows_v, sem):
        c = lax.axis_index("c")
        s = lax.axis_index("s")
        wid = s * NC + c
        base = wid * per_w

        def run(idx_h, out_h):
            def step(i, carry):
                off = base + i * CH
                pltpu.sync_copy(idx_h.at[pl.ds(off, CH)], idx_v)
                pltpu.async_copy(table_h.at[idx_v], rows_v, sem).wait()
                pltpu.sync_copy(rows_v, out_h.at[pl.ds(off, CH)])
                return carry
            lax.fori_loop(0, n_full, step, 0)
            if tail:
                off = base + n_full * CH
                pltpu.sync_copy(idx_h.at[pl.ds(off, tail)],
                                idx_v.at[pl.ds(0, tail)])
                pltpu.async_copy(table_h.at[idx_v.at[pl.ds(0, tail)]],
                                 rows_v.at[pl.ds(0, tail)], sem).wait()
                pltpu.sync_copy(rows_v.at[pl.ds(0, tail)],
                                out_h.at[pl.ds(off, tail)])

        run(src_h, srows_h)
        run(dst_h, drows_h)

    return pl.kernel(
        body,
        out_type=(jax.ShapeDtypeStruct((E, WD), jnp.float32),
                  jax.ShapeDtypeStruct((E, WD), jnp.float32)),
        mesh=mesh,
        scratch_types=[
            pltpu.VMEM((CH,), jnp.int32),
            pltpu.VMEM((CH, WD), jnp.float32),
            pltpu.SemaphoreType.DMA,
        ],
        compiler_params=pltpu.CompilerParams(use_tc_tiling_on_sc=False),
        name="egnn_sc_gather",
    )


@functools.lru_cache(maxsize=None)
def _make_scatter(N, E):
    per_w = E // NW
    n_full = per_w // CH
    tail = per_w - n_full * CH
    rows_t = N // NS
    mesh = plsc.VectorSubcoreMesh(core_axis_name="c", subcore_axis_name="s")

    def body(vals_h, dst_h, zeros_h, out_h, idx_v, idxt_v, vals_v, acc):
        c = lax.axis_index("c")
        s = lax.axis_index("s")
        wid = s * NC + c
        base = wid * per_w

        pltpu.sync_copy(zeros_h, acc.at[pl.ds(s * rows_t, rows_t)])
        plsc.subcore_barrier()

        def step(i, carry):
            off = base + i * CH
            pltpu.sync_copy(dst_h.at[pl.ds(off, CH)], idx_v)
            pltpu.sync_copy(vals_h.at[pl.ds(off, CH)], vals_v)
            pltpu.sync_copy(vals_v, acc.at[idx_v], add=True)
            return carry
        lax.fori_loop(0, n_full, step, 0)
        if tail:
            off = base + n_full * CH
            pltpu.sync_copy(dst_h.at[pl.ds(off, tail)], idxt_v)
            pltpu.sync_copy(vals_h.at[pl.ds(off, tail)],
                            vals_v.at[pl.ds(0, tail)])
            pltpu.sync_copy(vals_v.at[pl.ds(0, tail)], acc.at[idxt_v],
                            add=True)

        plsc.subcore_barrier()
        pltpu.sync_copy(acc.at[pl.ds(s * rows_t, rows_t)],
                        out_h.at[pl.ds(c * N + s * rows_t, rows_t)])

    return pl.kernel(
        body,
        out_type=jax.ShapeDtypeStruct((NC * N, WD), jnp.float32),
        mesh=mesh,
        scratch_types=[
            pltpu.VMEM((CH,), jnp.int32),
            pltpu.VMEM((max(tail, 8),), jnp.int32),
            pltpu.VMEM((CH, WD), jnp.float32),
            pltpu.VMEM_SHARED((N, WD), jnp.float32),
        ],
        compiler_params=pltpu.CompilerParams(use_tc_tiling_on_sc=False),
        name="egnn_sc_scatter",
    )


def _embed_pack(x, pos, embed_W, embed_b):
    N, F = x.shape
    BN = 2000

    def body(x_ref, p_ref, w_ref, b_ref, o_ref):
        feats = jnp.dot(x_ref[...], w_ref[...],
                        preferred_element_type=jnp.float32) + b_ref[...]
        o_ref[...] = jnp.concatenate(
            [feats, p_ref[...],
             jnp.zeros((BN, WD - H - 3), jnp.float32)], axis=1)

    return pl.pallas_call(
        body,
        grid=(N // BN,),
        in_specs=[
            pl.BlockSpec((BN, F), lambda i: (i, 0)),
            pl.BlockSpec((BN, 3), lambda i: (i, 0)),
            pl.BlockSpec((F, H), lambda i: (0, 0)),
            pl.BlockSpec((1, H), lambda i: (0, 0)),
        ],
        out_specs=pl.BlockSpec((BN, WD), lambda i: (i, 0)),
        out_shape=jax.ShapeDtypeStruct((N, WD), jnp.float32),
        name="egnn_embed_pack",
    )(x, pos, embed_W, embed_b.reshape(1, H))


def _edge_mlp(srows, drows, eattr, W1, b1, W2, b2, cW1, cb1, cW2, cb2):
    E = srows.shape[0]
    BE = 4000
    EI = W1.shape[0]
    EH = W1.shape[1]

    def body(s_ref, d_ref, a_ref, w1_ref, b1_ref, w2_ref, b2_ref,
             cw1_ref, cb1_ref, cw2_ref, cb2_ref, o_ref):
        sb = s_ref[...]
        db = d_ref[...]
        fs = sb[:, :H]
        fd = db[:, :H]
        rel = sb[:, H:H + 3] - db[:, H:H + 3]
        dist = jnp.sum(rel * rel, axis=1, keepdims=True)
        w1 = w1_ref[...]

        def b16(v):
            return v.astype(jnp.bfloat16).astype(jnp.float32)

        h = (jnp.dot(fd, w1[:H], preferred_element_type=jnp.float32)
             + jnp.dot(fs, w1[H:2 * H], preferred_element_type=jnp.float32)
             + b16(dist) * b16(w1[2 * H:2 * H + 1])
             + b16(a_ref[...]) * b16(w1[2 * H + 1:2 * H + 2])
             + b1_ref[...])
        h1 = _silu(h)
        m = _silu(jnp.dot(h1, w2_ref[...],
                          preferred_element_type=jnp.float32) + b2_ref[...])
        chid = _silu(jnp.dot(m, cw1_ref[...],
                             preferred_element_type=jnp.float32) + cb1_ref[...])
        cw = jnp.dot(chid, cw2_ref[...],
                     preferred_element_type=jnp.float32) + cb2_ref[...]
        o_ref[...] = jnp.concatenate(
            [m, cw * rel, jnp.zeros((BE, WD - H - 3), jnp.float32)], axis=1)

    full = lambda i: (0, 0)
    return pl.pallas_call(
        body,
        grid=(E // BE,),
        in_specs=[
            pl.BlockSpec((BE, WD), lambda i: (i, 0)),
            pl.BlockSpec((BE, WD), lambda i: (i, 0)),
            pl.BlockSpec((BE, 1), lambda i: (i, 0)),
            pl.BlockSpec((EI, EH), full),
            pl.BlockSpec((1, EH), full),
            pl.BlockSpec((EH, H), full),
            pl.BlockSpec((1, H), full),
            pl.BlockSpec((H, 4 * H), full),
            pl.BlockSpec((1, 4 * H), full),
            pl.BlockSpec((4 * H, 1), full),
            pl.BlockSpec((1, 1), full),
        ],
        out_specs=pl.BlockSpec((BE, WD), lambda i: (i, 0)),
        out_shape=jax.ShapeDtypeStruct((E, WD), jnp.float32),
        name="egnn_edge_mlp",
    )(srows, drows, eattr, W1, b1.reshape(1, EH), W2, b2.reshape(1, H),
      cW1, cb1.reshape(1, 4 * H), cW2, cb2.reshape(1, 1))


def _node_mlp(table, parts, nW1, nb1, nW2, nb2):
    N = table.shape[0]
    BN = 2000
    NH = nW1.shape[1]

    def body(t_ref, p0_ref, p1_ref, w1_ref, b1_ref, w2_ref, b2_ref, o_ref):
        tb = t_ref[...]
        feats = tb[:, :H]
        coors = tb[:, H:H + 3]
        m_i = p0_ref[:, :H] + p1_ref[:, :H]
        cdelta = p0_ref[:, H:H + 3] + p1_ref[:, H:H + 3]
        w1 = w1_ref[...]
        hmid = _silu(jnp.dot(feats, w1[:H], preferred_element_type=jnp.float32)
                     + jnp.dot(m_i, w1[H:2 * H],
                               preferred_element_type=jnp.float32)
                     + b1_ref[...])
        feats_new = feats + jnp.dot(hmid, w2_ref[...],
                                    preferred_element_type=jnp.float32) \
            + b2_ref[...]
        coors_new = coors + cdelta
        o_ref[...] = jnp.concatenate(
            [feats_new, coors_new,
             jnp.zeros((BN, WD - H - 3), jnp.float32)], axis=1)

    nb = N // BN
    full = lambda i: (0, 0)
    return pl.pallas_call(
        body,
        grid=(nb,),
        in_specs=[
            pl.BlockSpec((BN, WD), lambda i: (i, 0)),
            pl.BlockSpec((BN, WD), lambda i: (i, 0)),
            pl.BlockSpec((BN, WD), lambda i, nb=nb: (i + nb, 0)),
            pl.BlockSpec((2 * H, NH), full),
            pl.BlockSpec((1, NH), full),
            pl.BlockSpec((NH, H), full),
            pl.BlockSpec((1, H), full),
        ],
        out_specs=pl.BlockSpec((BN, WD), lambda i: (i, 0)),
        out_shape=jax.ShapeDtypeStruct((N, WD), jnp.float32),
        name="egnn_node_mlp",
    )(table, parts, parts, nW1, nb1.reshape(1, NH), nW2, nb2.reshape(1, H))


def _final_lin(table, lin_W, lin_b):
    N = table.shape[0]
    BN = 2000
    C = lin_W.shape[1]

    def body(t_ref, w_ref, b_ref, o_ref):
        o_ref[...] = jnp.dot(t_ref[:, :H], w_ref[...],
                             preferred_element_type=jnp.float32) + b_ref[...]

    return pl.pallas_call(
        body,
        grid=(N // BN,),
        in_specs=[
            pl.BlockSpec((BN, WD), lambda i: (i, 0)),
            pl.BlockSpec((H, C), lambda i: (0, 0)),
            pl.BlockSpec((1, C), lambda i: (0, 0)),
        ],
        out_specs=pl.BlockSpec((BN, C), lambda i: (i, 0)),
        out_shape=jax.ShapeDtypeStruct((N, C), jnp.float32),
        name="egnn_final_lin",
    )(table, lin_W, lin_b.reshape(1, C))


def kernel(x, edge_index, edge_attr, pos, embed_W, embed_b,
           edge_W1, edge_b1, edge_W2, edge_b2,
           coors_W1, coors_b1, coors_W2, coors_b2,
           node_W1, node_b1, node_W2, node_b2, lin_W, lin_b):
    N = x.shape[0]
    E = edge_index.shape[1]
    L = edge_W1.shape[0]
    assert E % NW == 0 and N % NS == 0

    src = edge_index[0]
    dst = edge_index[1]
    zeros_h = jnp.zeros((N // NS, WD), jnp.float32)

    gather = _make_gather(N, E)
    scatter = _make_scatter(N, E)

    table = _embed_pack(x, pos, embed_W, embed_b)
    for l in range(L):
        srows, drows = gather(table, src, dst)
        evals = _edge_mlp(srows, drows, edge_attr,
                          edge_W1[l], edge_b1[l], edge_W2[l], edge_b2[l],
                          coors_W1[l], coors_b1[l], coors_W2[l], coors_b2[l])
        parts = scatter(evals, dst, zeros_h)
        table = _node_mlp(table, parts, node_W1[l], node_b1[l],
                          node_W2[l], node_b2[l])
    return _final_lin(table, lin_W, lin_b)


# speedup vs baseline: 4.8705x; 1.4623x over previous
import functools

import jax
import jax.numpy as jnp
from jax import lax
from jax.experimental import pallas as pl
from jax.experimental.pallas import tpu as pltpu
from jax.experimental.pallas import tpu_sc as plsc

NC = 2
NS = 16
NW = NC * NS
CH = 128

WD = 32
H = 16


def _silu(v):
    return v * jax.nn.sigmoid(v)


@functools.lru_cache(maxsize=None)
def _make_gather(N, E):
    per_w = E // NW
    n_full = per_w // CH
    tail = per_w - n_full * CH
    mesh = plsc.VectorSubcoreMesh(core_axis_name="c", subcore_axis_name="s")

    def body(table_h, src_h, dst_h, srows_h, drows_h, idx_v, rows_v, sem):
        c = lax.axis_index("c")
        s = lax.axis_index("s")
        wid = s * NC + c
        base = wid * per_w

        def run(idx_h, out_h):
            def step(i, carry):
                off = base + i * CH
                pltpu.sync_copy(idx_h.at[pl.ds(off, CH)], idx_v)
                pltpu.async_copy(table_h.at[idx_v], rows_v, sem).wait()
                pltpu.sync_copy(rows_v, out_h.at[pl.ds(off, CH)])
                return carry
            lax.fori_loop(0, n_full, step, 0)
            if tail:
                off = base + n_full * CH
                pltpu.sync_copy(idx_h.at[pl.ds(off, tail)],
                                idx_v.at[pl.ds(0, tail)])
                pltpu.async_copy(table_h.at[idx_v.at[pl.ds(0, tail)]],
                                 rows_v.at[pl.ds(0, tail)], sem).wait()
                pltpu.sync_copy(rows_v.at[pl.ds(0, tail)],
                                out_h.at[pl.ds(off, tail)])

        run(src_h, srows_h)
        run(dst_h, drows_h)

    return pl.kernel(
        body,
        out_type=(jax.ShapeDtypeStruct((E, WD), jnp.float32),
                  jax.ShapeDtypeStruct((E, WD), jnp.float32)),
        mesh=mesh,
        scratch_types=[
            pltpu.VMEM((CH,), jnp.int32),
            pltpu.VMEM((CH, WD), jnp.float32),
            pltpu.SemaphoreType.DMA,
        ],
        compiler_params=pltpu.CompilerParams(use_tc_tiling_on_sc=False),
        name="egnn_sc_gather",
    )


@functools.lru_cache(maxsize=None)
def _make_scatter(N, E):
    per_w = E // NW
    n_full = per_w // CH
    tail = per_w - n_full * CH
    rows_t = N // NS
    mesh = plsc.VectorSubcoreMesh(core_axis_name="c", subcore_axis_name="s")

    def body(vals_h, dst_h, zeros_h, out_h, idx_v, idxt_v, vals_v, acc):
        c = lax.axis_index("c")
        s = lax.axis_index("s")
        wid = s * NC + c
        base = wid * per_w

        pltpu.sync_copy(zeros_h, acc.at[pl.ds(s * rows_t, rows_t)])
        plsc.subcore_barrier()

        def step(i, carry):
            off = base + i * CH
            pltpu.sync_copy(dst_h.at[pl.ds(off, CH)], idx_v)
            pltpu.sync_copy(vals_h.at[pl.ds(off, CH)], vals_v)
            pltpu.sync_copy(vals_v, acc.at[idx_v], add=True)
            return carry
        lax.fori_loop(0, n_full, step, 0)
        if tail:
            off = base + n_full * CH
            pltpu.sync_copy(dst_h.at[pl.ds(off, tail)], idxt_v)
            pltpu.sync_copy(vals_h.at[pl.ds(off, tail)],
                            vals_v.at[pl.ds(0, tail)])
            pltpu.sync_copy(vals_v.at[pl.ds(0, tail)], acc.at[idxt_v],
                            add=True)

        plsc.subcore_barrier()
        pltpu.sync_copy(acc.at[pl.ds(s * rows_t, rows_t)],
                        out_h.at[pl.ds(c * N + s * rows_t, rows_t)])

    return pl.kernel(
        body,
        out_type=jax.ShapeDtypeStruct((NC * N, WD), jnp.float32),
        mesh=mesh,
        scratch_types=[
            pltpu.VMEM((CH,), jnp.int32),
            pltpu.VMEM((max(tail, 8),), jnp.int32),
            pltpu.VMEM((CH, WD), jnp.float32),
            pltpu.VMEM_SHARED((N, WD), jnp.float32),
        ],
        compiler_params=pltpu.CompilerParams(use_tc_tiling_on_sc=False),
        name="egnn_sc_scatter",
    )


def _embed_pack(x, pos, embed_W, embed_b):
    N, F = x.shape
    BN = 2000

    def body(x_ref, p_ref, w_ref, b_ref, o_ref):
        feats = jnp.dot(x_ref[...], w_ref[...],
                        preferred_element_type=jnp.float32) + b_ref[...]
        o_ref[...] = jnp.concatenate(
            [feats, p_ref[...],
             jnp.zeros((BN, WD - H - 3), jnp.float32)], axis=1)

    return pl.pallas_call(
        body,
        grid=(N // BN,),
        in_specs=[
            pl.BlockSpec((BN, F), lambda i: (i, 0)),
            pl.BlockSpec((BN, 3), lambda i: (i, 0)),
            pl.BlockSpec((F, H), lambda i: (0, 0)),
            pl.BlockSpec((1, H), lambda i: (0, 0)),
        ],
        out_specs=pl.BlockSpec((BN, WD), lambda i: (i, 0)),
        out_shape=jax.ShapeDtypeStruct((N, WD), jnp.float32),
        name="egnn_embed_pack",
    )(x, pos, embed_W, embed_b.reshape(1, H))


SL = 4
VW = SL * WD


def _bdiag(w, rstep, cstep, roff=0):
    out = jnp.zeros((SL * rstep, SL * cstep), jnp.float32)
    for j in range(SL):
        out = out.at[j * rstep + roff:j * rstep + roff + w.shape[0],
                     j * cstep:j * cstep + w.shape[1]].set(w)
    return out


def _edge_mlp(sview, dview, attrs, W1, b1, W2, b2, cW1, cb1, cW2, cb2):
    EV = sview.shape[0]
    BV = 1600
    EH = W1.shape[1]

    wd_blk = _bdiag(W1[:H], WD, EH)
    ws_blk = _bdiag(W1[H:2 * H], WD, EH)
    w2_blk = _bdiag(W2, EH, H)
    cw1_blk = _bdiag(cW1, H, 4 * H)
    cw2_blk = _bdiag(cW2, 4 * H, 1)
    b1t = jnp.tile(b1.reshape(1, EH), (1, SL))
    b2t = jnp.tile(b2.reshape(1, H), (1, SL))
    cb1t = jnp.tile(cb1.reshape(1, 4 * H), (1, SL))
    cb2t = jnp.tile(cb2.reshape(1, 1), (1, SL))
    w1d = W1[2 * H:2 * H + 1]
    w1a = W1[2 * H + 1:2 * H + 2]

    def body(s_ref, d_ref, a0_ref, a1_ref, a2_ref, a3_ref,
             wd_ref, ws_ref, w2_ref, cw1_ref, cw2_ref,
             b1_ref, b2_ref, cb1_ref, cb2_ref, w1d_ref, w1a_ref, o_ref):
        sb = s_ref[...]
        db = d_ref[...]

        def b16(v):
            return v.astype(jnp.bfloat16).astype(jnp.float32)

        h = (jnp.dot(db, wd_ref[...], preferred_element_type=jnp.float32)
             + jnp.dot(sb, ws_ref[...], preferred_element_type=jnp.float32)
             + b1_ref[...])
        a_refs = (a0_ref, a1_ref, a2_ref, a3_ref)
        rels = []
        corrs = []
        for j in range(SL):
            c0 = j * WD + H
            rel_j = sb[:, c0:c0 + 3] - db[:, c0:c0 + 3]
            dist_j = jnp.sum(rel_j * rel_j, axis=1, keepdims=True)
            corrs.append(b16(dist_j) * b16(w1d_ref[...])
                         + b16(a_refs[j][...]) * b16(w1a_ref[...]))
            rels.append(rel_j)
        h = h + jnp.concatenate(corrs, axis=1)
        h1 = _silu(h)
        m_all = _silu(jnp.dot(h1, w2_ref[...],
                              preferred_element_type=jnp.float32) + b2_ref[...])
        chid = _silu(jnp.dot(m_all, cw1_ref[...],
                             preferred_element_type=jnp.float32) + cb1_ref[...])
        cw_all = jnp.dot(chid, cw2_ref[...],
                         preferred_element_type=jnp.float32) + cb2_ref[...]
        pieces = []
        for j in range(SL):
            pieces.append(m_all[:, j * H:(j + 1) * H])
            pieces.append(cw_all[:, j:j + 1] * rels[j])
            pieces.append(jnp.zeros((BV, WD - H - 3), jnp.float32))
        o_ref[...] = jnp.concatenate(pieces, axis=1)

    full = lambda i: (0, 0)
    blk = lambda i: (i, 0)
    return pl.pallas_call(
        body,
        grid=(EV // BV,),
        in_specs=[
            pl.BlockSpec((BV, VW), blk),
            pl.BlockSpec((BV, VW), blk),
            pl.BlockSpec((BV, 1), blk),
            pl.BlockSpec((BV, 1), blk),
            pl.BlockSpec((BV, 1), blk),
            pl.BlockSpec((BV, 1), blk),
            pl.BlockSpec(wd_blk.shape, full),
            pl.BlockSpec(ws_blk.shape, full),
            pl.BlockSpec(w2_blk.shape, full),
            pl.BlockSpec(cw1_blk.shape, full),
            pl.BlockSpec(cw2_blk.shape, full),
            pl.BlockSpec(b1t.shape, full),
            pl.BlockSpec(b2t.shape, full),
            pl.BlockSpec(cb1t.shape, full),
            pl.BlockSpec(cb2t.shape, full),
            pl.BlockSpec(w1d.shape, full),
            pl.BlockSpec(w1a.shape, full),
        ],
        out_specs=pl.BlockSpec((BV, VW), blk),
        out_shape=jax.ShapeDtypeStruct((EV, VW), jnp.float32),
        name="egnn_edge_mlp",
    )(sview, dview, attrs[0], attrs[1], attrs[2], attrs[3],
      wd_blk, ws_blk, w2_blk, cw1_blk, cw2_blk,
      b1t, b2t, cb1t, cb2t, w1d, w1a)


def _node_mlp(tview, pview, nW1, nb1, nW2, nb2):
    NV = tview.shape[0]
    BV = NV
    NH = nW1.shape[1]

    tw_blk = _bdiag(nW1[:H], WD, NH)
    pw_blk = _bdiag(nW1[H:2 * H], WD, NH)
    w2_blk = _bdiag(nW2, NH, H)
    b1t = jnp.tile(nb1.reshape(1, NH), (1, SL))
    b2t = jnp.tile(nb2.reshape(1, H), (1, SL))
    nb_blocks = NV // BV

    def body(t_ref, p_ref, tw_ref, pw_ref, w2_ref,
             b1_ref, b2_ref, o_ref):
        tb = t_ref[...]
        pb = p_ref[0] + p_ref[1]
        hmid = _silu(jnp.dot(tb, tw_ref[...],
                             preferred_element_type=jnp.float32)
                     + jnp.dot(pb, pw_ref[...],
                               preferred_element_type=jnp.float32)
                     + b1_ref[...])
        fdel = jnp.dot(hmid, w2_ref[...],
                       preferred_element_type=jnp.float32) + b2_ref[...]
        pieces = []
        for j in range(SL):
            c0 = j * WD
            pieces.append(tb[:, c0:c0 + H] + fdel[:, j * H:(j + 1) * H])
            pieces.append(tb[:, c0 + H:c0 + H + 3] + pb[:, c0 + H:c0 + H + 3])
            pieces.append(jnp.zeros((BV, WD - H - 3), jnp.float32))
        o_ref[...] = jnp.concatenate(pieces, axis=1)

    full = lambda i: (0, 0)
    blk = lambda i: (i, 0)
    return pl.pallas_call(
        body,
        grid=(nb_blocks,),
        in_specs=[
            pl.BlockSpec((BV, VW), blk),
            pl.BlockSpec((2, BV, VW), lambda i: (0, i, 0)),
            pl.BlockSpec(tw_blk.shape, full),
            pl.BlockSpec(pw_blk.shape, full),
            pl.BlockSpec(w2_blk.shape, full),
            pl.BlockSpec(b1t.shape, full),
            pl.BlockSpec(b2t.shape, full),
        ],
        out_specs=pl.BlockSpec((BV, VW), blk),
        out_shape=jax.ShapeDtypeStruct((NV, VW), jnp.float32),
        name="egnn_node_mlp",
    )(tview, pview.reshape(2, NV, VW), tw_blk, pw_blk, w2_blk, b1t, b2t)


def _final_lin(tview, lin_W, lin_b):
    NV = tview.shape[0]
    BV = NV
    C = lin_W.shape[1]

    lin_blk = _bdiag(lin_W, WD, C)
    bt = jnp.tile(lin_b.reshape(1, C), (1, SL))

    def body(t_ref, w_ref, b_ref, o_ref):
        o_ref[...] = jnp.dot(t_ref[...], w_ref[...],
                             preferred_element_type=jnp.float32) + b_ref[...]

    return pl.pallas_call(
        body,
        grid=(NV // BV,),
        in_specs=[
            pl.BlockSpec((BV, VW), lambda i: (i, 0)),
            pl.BlockSpec(lin_blk.shape, lambda i: (0, 0)),
            pl.BlockSpec(bt.shape, lambda i: (0, 0)),
        ],
        out_specs=pl.BlockSpec((BV, SL * C), lambda i: (i, 0)),
        out_shape=jax.ShapeDtypeStruct((NV, SL * C), jnp.float32),
        name="egnn_final_lin",
    )(tview, lin_blk, bt)


def kernel(x, edge_index, edge_attr, pos, embed_W, embed_b,
           edge_W1, edge_b1, edge_W2, edge_b2,
           coors_W1, coors_b1, coors_W2, coors_b2,
           node_W1, node_b1, node_W2, node_b2, lin_W, lin_b):
    N = x.shape[0]
    E = edge_index.shape[1]
    L = edge_W1.shape[0]
    assert E % NW == 0 and N % NS == 0

    src = edge_index[0]
    dst = edge_index[1]
    zeros_h = jnp.zeros((N // NS, WD), jnp.float32)
    attrs = [edge_attr[j::SL] for j in range(SL)]

    gather = _make_gather(N, E)
    scatter = _make_scatter(N, E)

    table = _embed_pack(x, pos, embed_W, embed_b)
    for l in range(L):
        srows, drows = gather(table, src, dst)
        evals_v = _edge_mlp(srows.reshape(E // SL, VW),
                            drows.reshape(E // SL, VW), attrs,
                            edge_W1[l], edge_b1[l], edge_W2[l], edge_b2[l],
                            coors_W1[l], coors_b1[l], coors_W2[l], coors_b2[l])
        parts = scatter(evals_v.reshape(E, WD), dst, zeros_h)
        tview = _node_mlp(table.reshape(N // SL, VW),
                          parts.reshape(2 * N // SL, VW),
                          node_W1[l], node_b1[l], node_W2[l], node_b2[l])
        table = tview.reshape(N, WD)
    out_v = _final_lin(table.reshape(N // SL, VW), lin_W, lin_b)
    return out_v.reshape(N, lin_W.shape[1])


# speedup vs baseline: 7.5907x; 1.5585x over previous
import functools

import jax
import jax.numpy as jnp
from jax import lax
from jax.experimental import pallas as pl
from jax.experimental.pallas import tpu as pltpu
from jax.experimental.pallas import tpu_sc as plsc

NC = 2
NS = 16
NW = NC * NS
CH = 128

WD = 32
H = 16


def _silu(v):
    return v * jax.nn.sigmoid(v)


@functools.lru_cache(maxsize=None)
def _make_gather(N, E):
    per_w = E // NW
    n_full = per_w // CH
    tail = per_w - n_full * CH
    mesh = plsc.VectorSubcoreMesh(core_axis_name="c", subcore_axis_name="s")

    D = 6
    K = 3
    n_main = (n_full // D) * D

    def body(table_h, src_h, dst_h, srows_h, drows_h, *scr):
        c = lax.axis_index("c")
        s = lax.axis_index("s")
        wid = s * NC + c
        base = wid * per_w

        idxs = scr[0:2]
        rows = (scr[2:2 + D], scr[2 + D:2 + 2 * D])
        gsem = (scr[2 + 2 * D:2 + 3 * D], scr[2 + 3 * D:2 + 4 * D])
        ssem = (scr[2 + 4 * D:2 + 5 * D], scr[2 + 5 * D:2 + 6 * D])
        sem_i = scr[2 + 6 * D]
        idxs_v, idxd_v = idxs
        outs = (srows_h, drows_h)

        pltpu.async_copy(src_h.at[pl.ds(base, per_w)], idxs_v, sem_i).wait()
        pltpu.async_copy(dst_h.at[pl.ds(base, per_w)], idxd_v, sem_i).wait()

        def g_start(h, p, i):
            return pltpu.async_copy(
                table_h.at[idxs[h].at[pl.ds(i * CH, CH)]], rows[h][p],
                gsem[h][p])

        def g_wait(h, p, i):
            pltpu.make_async_copy(
                table_h.at[idxs[h].at[pl.ds(i * CH, CH)]], rows[h][p],
                gsem[h][p]).wait()

        def s_start(h, p, i):
            return pltpu.async_copy(
                rows[h][p], outs[h].at[pl.ds(base + i * CH, CH)], ssem[h][p])

        def s_wait(h, p, i):
            pltpu.make_async_copy(
                rows[h][p], outs[h].at[pl.ds(base + i * CH, CH)],
                ssem[h][p]).wait()

        for h in (0, 1):
            for p in range(K):
                g_start(h, p, p)

        def step_grp(g, carry):
            for p in range(D):
                i = g * D + p
                for h in (0, 1):
                    pf = (p + K) % D

                    @pl.when(i + K < n_full)
                    def _(h=h, pf=pf, i=i):
                        @pl.when(i + K >= D)
                        def _():
                            s_wait(h, pf, i + K - D)
                        g_start(h, pf, i + K)

                    g_wait(h, p, i)
                    s_start(h, p, i)
            return carry
        lax.fori_loop(0, n_main // D, step_grp, 0)

        assert n_full - n_main <= K
        for i in range(n_main, n_full):
            p = i % D
            for h in (0, 1):
                g_wait(h, p, i)
                s_start(h, p, i)

        if tail:
            t = n_full % D
            off = base + n_full * CH
            for h in (0, 1):
                s_wait(h, t, n_full - D)
                src_sl = table_h.at[idxs[h].at[pl.ds(n_full * CH, tail)]]
                dst_sl = rows[h][t].at[pl.ds(0, tail)]
                pltpu.async_copy(src_sl, dst_sl, gsem[h][t]).wait()
                pltpu.async_copy(dst_sl, outs[h].at[pl.ds(off, tail)],
                                 ssem[h][t]).wait()

        for i in range(n_full - D, n_full):
            p = i % D
            if tail and p == n_full % D:
                continue
            for h in (0, 1):
                s_wait(h, p, i)

    sems = [pltpu.SemaphoreType.DMA] * (4 * D + 1)
    return pl.kernel(
        body,
        out_type=(jax.ShapeDtypeStruct((E, WD), jnp.float32),
                  jax.ShapeDtypeStruct((E, WD), jnp.float32)),
        mesh=mesh,
        scratch_types=(
            [pltpu.VMEM((per_w,), jnp.int32)] * 2
            + [pltpu.VMEM((CH, WD), jnp.float32)] * (2 * D)
            + sems
        ),
        compiler_params=pltpu.CompilerParams(use_tc_tiling_on_sc=False),
        name="egnn_sc_gather",
    )


@functools.lru_cache(maxsize=None)
def _make_scatter(N, E):
    per_w = E // NW
    n_full = per_w // CH
    tail = per_w - n_full * CH
    rows_t = N // NS
    mesh = plsc.VectorSubcoreMesh(core_axis_name="c", subcore_axis_name="s")

    D = 4
    LA = 2
    n_main = (n_full // D) * D
    fired_max = n_main - 1 + LA

    def body(vals_h, dst_h, zeros_h, out_h, *scr):
        c = lax.axis_index("c")
        s = lax.axis_index("s")
        wid = s * NC + c
        base = wid * per_w

        idxb = scr[0:D]
        idxt = scr[D]
        valb = scr[D + 1:2 * D + 1]
        acc = scr[2 * D + 1]
        six = scr[2 * D + 2:3 * D + 2]
        sv = scr[3 * D + 2:4 * D + 2]
        sa = scr[4 * D + 2:5 * D + 2]

        pltpu.sync_copy(zeros_h, acc.at[pl.ds(s * rows_t, rows_t)])
        plsc.subcore_barrier()

        def ld_start(p, i):
            pltpu.async_copy(dst_h.at[pl.ds(base + i * CH, CH)], idxb[p],
                             six[p])
            pltpu.async_copy(vals_h.at[pl.ds(base + i * CH, CH)], valb[p],
                             sv[p])

        def ld_wait(p, i):
            pltpu.make_async_copy(dst_h.at[pl.ds(base + i * CH, CH)],
                                  idxb[p], six[p]).wait()
            pltpu.make_async_copy(vals_h.at[pl.ds(base + i * CH, CH)],
                                  valb[p], sv[p]).wait()

        def a_start(p):
            pltpu.async_copy(valb[p], acc.at[idxb[p]], sa[p], add=True)

        def a_wait(p):
            pltpu.make_async_copy(valb[p], acc.at[idxb[p]], sa[p]).wait()

        for k in range(LA):
            ld_start(k % D, k)

        def step_grp(g, carry):
            for p in range(D):
                i = g * D + p
                q = (p + LA) % D

                @pl.when(i + LA < n_full)
                def _(p=p, q=q, i=i):
                    @pl.when(i + LA >= D)
                    def _():
                        a_wait(q)
                    ld_start(q, i + LA)

                ld_wait(p, i)
                a_start(p)
            return carry
        lax.fori_loop(0, n_main // D, step_grp, 0)

        for i in range(n_main, n_full):
            p = i % D
            if i > fired_max:
                a_wait(p)
                ld_start(p, i)
            ld_wait(p, i)
            a_start(p)

        if tail:
            tp = n_full % D
            off = base + n_full * CH
            a_wait(tp)
            pltpu.sync_copy(dst_h.at[pl.ds(off, tail)], idxt)
            pltpu.sync_copy(vals_h.at[pl.ds(off, tail)],
                            valb[tp].at[pl.ds(0, tail)])
            pltpu.sync_copy(valb[tp].at[pl.ds(0, tail)], acc.at[idxt],
                            add=True)

        for i in range(n_full - D, n_full):
            p = i % D
            if tail and p == n_full % D:
                continue
            a_wait(p)

        plsc.subcore_barrier()
        pltpu.sync_copy(acc.at[pl.ds(s * rows_t, rows_t)],
                        out_h.at[pl.ds(c * N + s * rows_t, rows_t)])

    return pl.kernel(
        body,
        out_type=jax.ShapeDtypeStruct((NC * N, WD), jnp.float32),
        mesh=mesh,
        scratch_types=(
            [pltpu.VMEM((CH,), jnp.int32)] * D
            + [pltpu.VMEM((max(tail, 8),), jnp.int32)]
            + [pltpu.VMEM((CH, WD), jnp.float32)] * D
            + [pltpu.VMEM_SHARED((N, WD), jnp.float32)]
            + [pltpu.SemaphoreType.DMA] * (3 * D)
        ),
        compiler_params=pltpu.CompilerParams(use_tc_tiling_on_sc=False),
        name="egnn_sc_scatter",
    )


def _embed_pack(x, pos, embed_W, embed_b):
    N, F = x.shape
    BN = 2000

    def body(x_ref, p_ref, w_ref, b_ref, o_ref):
        feats = jnp.dot(x_ref[...], w_ref[...],
                        preferred_element_type=jnp.float32) + b_ref[...]
        o_ref[...] = jnp.concatenate(
            [feats, p_ref[...],
             jnp.zeros((BN, WD - H - 3), jnp.float32)], axis=1)

    return pl.pallas_call(
        body,
        grid=(N // BN,),
        in_specs=[
            pl.BlockSpec((BN, F), lambda i: (i, 0)),
            pl.BlockSpec((BN, 3), lambda i: (i, 0)),
            pl.BlockSpec((F, H), lambda i: (0, 0)),
            pl.BlockSpec((1, H), lambda i: (0, 0)),
        ],
        out_specs=pl.BlockSpec((BN, WD), lambda i: (i, 0)),
        out_shape=jax.ShapeDtypeStruct((N, WD), jnp.float32),
        name="egnn_embed_pack",
    )(x, pos, embed_W, embed_b.reshape(1, H))


SL = 4
VW = SL * WD


def _bdiag(w, rstep, cstep, roff=0):
    out = jnp.zeros((SL * rstep, SL * cstep), jnp.float32)
    for j in range(SL):
        out = out.at[j * rstep + roff:j * rstep + roff + w.shape[0],
                     j * cstep:j * cstep + w.shape[1]].set(w)
    return out


def _edge_mlp(sview, dview, attrs, W1, b1, W2, b2, cW1, cb1, cW2, cb2):
    EV = sview.shape[0]
    BV = 1600
    EH = W1.shape[1]

    wd_blk = _bdiag(W1[:H], WD, EH)
    ws_blk = _bdiag(W1[H:2 * H], WD, EH)
    w2_blk = _bdiag(W2, EH, H)
    cw1_blk = _bdiag(cW1, H, 4 * H)
    cw2_blk = _bdiag(cW2, 4 * H, 1)
    b1t = jnp.tile(b1.reshape(1, EH), (1, SL))
    b2t = jnp.tile(b2.reshape(1, H), (1, SL))
    cb1t = jnp.tile(cb1.reshape(1, 4 * H), (1, SL))
    cb2t = jnp.tile(cb2.reshape(1, 1), (1, SL))
    w1d = W1[2 * H:2 * H + 1]
    w1a = W1[2 * H + 1:2 * H + 2]

    def body(s_ref, d_ref, a0_ref, a1_ref, a2_ref, a3_ref,
             wd_ref, ws_ref, w2_ref, cw1_ref, cw2_ref,
             b1_ref, b2_ref, cb1_ref, cb2_ref, w1d_ref, w1a_ref, o_ref):
        sb = s_ref[...]
        db = d_ref[...]

        def b16(v):
            return v.astype(jnp.bfloat16).astype(jnp.float32)

        h = (jnp.dot(db, wd_ref[...], preferred_element_type=jnp.float32)
             + jnp.dot(sb, ws_ref[...], preferred_element_type=jnp.float32)
             + b1_ref[...])
        a_refs = (a0_ref, a1_ref, a2_ref, a3_ref)
        rels = []
        corrs = []
        for j in range(SL):
            c0 = j * WD + H
            rel_j = sb[:, c0:c0 + 3] - db[:, c0:c0 + 3]
            dist_j = jnp.sum(rel_j * rel_j, axis=1, keepdims=True)
            corrs.append(b16(dist_j) * b16(w1d_ref[...])
                         + b16(a_refs[j][...]) * b16(w1a_ref[...]))
            rels.append(rel_j)
        h = h + jnp.concatenate(corrs, axis=1)
        h1 = _silu(h)
        m_all = _silu(jnp.dot(h1, w2_ref[...],
                              preferred_element_type=jnp.float32) + b2_ref[...])
        chid = _silu(jnp.dot(m_all, cw1_ref[...],
                             preferred_element_type=jnp.float32) + cb1_ref[...])
        cw_all = jnp.dot(chid, cw2_ref[...],
                         preferred_element_type=jnp.float32) + cb2_ref[...]
        pieces = []
        for j in range(SL):
            pieces.append(m_all[:, j * H:(j + 1) * H])
            pieces.append(cw_all[:, j:j + 1] * rels[j])
            pieces.append(jnp.zeros((BV, WD - H - 3), jnp.float32))
        o_ref[...] = jnp.concatenate(pieces, axis=1)

    full = lambda i: (0, 0)
    blk = lambda i: (i, 0)
    return pl.pallas_call(
        body,
        grid=(EV // BV,),
        in_specs=[
            pl.BlockSpec((BV, VW), blk),
            pl.BlockSpec((BV, VW), blk),
            pl.BlockSpec((BV, 1), blk),
            pl.BlockSpec((BV, 1), blk),
            pl.BlockSpec((BV, 1), blk),
            pl.BlockSpec((BV, 1), blk),
            pl.BlockSpec(wd_blk.shape, full),
            pl.BlockSpec(ws_blk.shape, full),
            pl.BlockSpec(w2_blk.shape, full),
            pl.BlockSpec(cw1_blk.shape, full),
            pl.BlockSpec(cw2_blk.shape, full),
            pl.BlockSpec(b1t.shape, full),
            pl.BlockSpec(b2t.shape, full),
            pl.BlockSpec(cb1t.shape, full),
            pl.BlockSpec(cb2t.shape, full),
            pl.BlockSpec(w1d.shape, full),
            pl.BlockSpec(w1a.shape, full),
        ],
        out_specs=pl.BlockSpec((BV, VW), blk),
        out_shape=jax.ShapeDtypeStruct((EV, VW), jnp.float32),
        name="egnn_edge_mlp",
    )(sview, dview, attrs[0], attrs[1], attrs[2], attrs[3],
      wd_blk, ws_blk, w2_blk, cw1_blk, cw2_blk,
      b1t, b2t, cb1t, cb2t, w1d, w1a)


def _node_mlp(tview, pview, nW1, nb1, nW2, nb2):
    NV = tview.shape[0]
    BV = NV
    NH = nW1.shape[1]

    tw_blk = _bdiag(nW1[:H], WD, NH)
    pw_blk = _bdiag(nW1[H:2 * H], WD, NH)
    w2_blk = _bdiag(nW2, NH, H)
    b1t = jnp.tile(nb1.reshape(1, NH), (1, SL))
    b2t = jnp.tile(nb2.reshape(1, H), (1, SL))
    nb_blocks = NV // BV

    def body(t_ref, p_ref, tw_ref, pw_ref, w2_ref,
             b1_ref, b2_ref, o_ref):
        tb = t_ref[...]
        pb = p_ref[0] + p_ref[1]
        hmid = _silu(jnp.dot(tb, tw_ref[...],
                             preferred_element_type=jnp.float32)
                     + jnp.dot(pb, pw_ref[...],
                               preferred_element_type=jnp.float32)
                     + b1_ref[...])
        fdel = jnp.dot(hmid, w2_ref[...],
                       preferred_element_type=jnp.float32) + b2_ref[...]
        pieces = []
        for j in range(SL):
            c0 = j * WD
            pieces.append(tb[:, c0:c0 + H] + fdel[:, j * H:(j + 1) * H])
            pieces.append(tb[:, c0 + H:c0 + H + 3] + pb[:, c0 + H:c0 + H + 3])
            pieces.append(jnp.zeros((BV, WD - H - 3), jnp.float32))
        o_ref[...] = jnp.concatenate(pieces, axis=1)

    full = lambda i: (0, 0)
    blk = lambda i: (i, 0)
    return pl.pallas_call(
        body,
        grid=(nb_blocks,),
        in_specs=[
            pl.BlockSpec((BV, VW), blk),
            pl.BlockSpec((2, BV, VW), lambda i: (0, i, 0)),
            pl.BlockSpec(tw_blk.shape, full),
            pl.BlockSpec(pw_blk.shape, full),
            pl.BlockSpec(w2_blk.shape, full),
            pl.BlockSpec(b1t.shape, full),
            pl.BlockSpec(b2t.shape, full),
        ],
        out_specs=pl.BlockSpec((BV, VW), blk),
        out_shape=jax.ShapeDtypeStruct((NV, VW), jnp.float32),
        name="egnn_node_mlp",
    )(tview, pview.reshape(2, NV, VW), tw_blk, pw_blk, w2_blk, b1t, b2t)


def _final_lin(tview, lin_W, lin_b):
    NV = tview.shape[0]
    BV = NV
    C = lin_W.shape[1]

    lin_blk = _bdiag(lin_W, WD, C)
    bt = jnp.tile(lin_b.reshape(1, C), (1, SL))

    def body(t_ref, w_ref, b_ref, o_ref):
        o_ref[...] = jnp.dot(t_ref[...], w_ref[...],
                             preferred_element_type=jnp.float32) + b_ref[...]

    return pl.pallas_call(
        body,
        grid=(NV // BV,),
        in_specs=[
            pl.BlockSpec((BV, VW), lambda i: (i, 0)),
            pl.BlockSpec(lin_blk.shape, lambda i: (0, 0)),
            pl.BlockSpec(bt.shape, lambda i: (0, 0)),
        ],
        out_specs=pl.BlockSpec((BV, SL * C), lambda i: (i, 0)),
        out_shape=jax.ShapeDtypeStruct((NV, SL * C), jnp.float32),
        name="egnn_final_lin",
    )(tview, lin_blk, bt)


def kernel(x, edge_index, edge_attr, pos, embed_W, embed_b,
           edge_W1, edge_b1, edge_W2, edge_b2,
           coors_W1, coors_b1, coors_W2, coors_b2,
           node_W1, node_b1, node_W2, node_b2, lin_W, lin_b):
    N = x.shape[0]
    E = edge_index.shape[1]
    L = edge_W1.shape[0]
    assert E % NW == 0 and N % NS == 0

    src = edge_index[0]
    dst = edge_index[1]
    zeros_h = jnp.zeros((N // NS, WD), jnp.float32)
    attrs = [edge_attr[j::SL] for j in range(SL)]

    gather = _make_gather(N, E)
    scatter = _make_scatter(N, E)

    table = _embed_pack(x, pos, embed_W, embed_b)
    for l in range(L):
        srows, drows = gather(table, src, dst)
        evals_v = _edge_mlp(srows.reshape(E // SL, VW),
                            drows.reshape(E // SL, VW), attrs,
                            edge_W1[l], edge_b1[l], edge_W2[l], edge_b2[l],
                            coors_W1[l], coors_b1[l], coors_W2[l], coors_b2[l])
        parts = scatter(evals_v.reshape(E, WD), dst, zeros_h)
        tview = _node_mlp(table.reshape(N // SL, VW),
                          parts.reshape(2 * N // SL, VW),
                          node_W1[l], node_b1[l], node_W2[l], node_b2[l])
        table = tview.reshape(N, WD)
    out_v = _final_lin(table.reshape(N // SL, VW), lin_W, lin_b)
    return out_v.reshape(N, lin_W.shape[1])


# speedup vs baseline: 10.7377x; 1.4146x over previous
import functools

import jax
import jax.numpy as jnp
from jax import lax
from jax.experimental import pallas as pl
from jax.experimental.pallas import tpu as pltpu
from jax.experimental.pallas import tpu_sc as plsc

NC = 2
NS = 16
NW = NC * NS
CH = 128

WD = 32
H = 16


def _silu(v):
    return v * jax.nn.sigmoid(v)


@functools.lru_cache(maxsize=None)
def _make_gather(N, E):
    per_w = E // NW
    n_full = per_w // CH
    tail = per_w - n_full * CH
    mesh = plsc.VectorSubcoreMesh(core_axis_name="c", subcore_axis_name="s")

    D = 6
    K = 3
    n_main = (n_full // D) * D

    def body(table_h, src_h, dst_h, srows_h, drows_h, *scr):
        c = lax.axis_index("c")
        s = lax.axis_index("s")
        wid = s * NC + c
        base = wid * per_w

        idxs = scr[0:2]
        rows = (scr[2:2 + D], scr[2 + D:2 + 2 * D])
        gsem = (scr[2 + 2 * D:2 + 3 * D], scr[2 + 3 * D:2 + 4 * D])
        ssem = (scr[2 + 4 * D:2 + 5 * D], scr[2 + 5 * D:2 + 6 * D])
        sem_i = scr[2 + 6 * D]
        idxs_v, idxd_v = idxs
        outs = (srows_h, drows_h)

        pltpu.async_copy(src_h.at[pl.ds(base, per_w)], idxs_v, sem_i).wait()
        pltpu.async_copy(dst_h.at[pl.ds(base, per_w)], idxd_v, sem_i).wait()

        def g_start(h, p, i):
            return pltpu.async_copy(
                table_h.at[idxs[h].at[pl.ds(i * CH, CH)]], rows[h][p],
                gsem[h][p])

        def g_wait(h, p, i):
            pltpu.make_async_copy(
                table_h.at[idxs[h].at[pl.ds(i * CH, CH)]], rows[h][p],
                gsem[h][p]).wait()

        def s_start(h, p, i):
            return pltpu.async_copy(
                rows[h][p], outs[h].at[pl.ds(base + i * CH, CH)], ssem[h][p])

        def s_wait(h, p, i):
            pltpu.make_async_copy(
                rows[h][p], outs[h].at[pl.ds(base + i * CH, CH)],
                ssem[h][p]).wait()

        for h in (0, 1):
            for p in range(K):
                g_start(h, p, p)

        def step_grp(g, carry):
            for p in range(D):
                i = g * D + p
                for h in (0, 1):
                    pf = (p + K) % D

                    @pl.when(i + K < n_full)
                    def _(h=h, pf=pf, i=i):
                        @pl.when(i + K >= D)
                        def _():
                            s_wait(h, pf, i + K - D)
                        g_start(h, pf, i + K)

                    g_wait(h, p, i)
                    s_start(h, p, i)
            return carry
        lax.fori_loop(0, n_main // D, step_grp, 0)

        assert n_full - n_main <= K
        for i in range(n_main, n_full):
            p = i % D
            for h in (0, 1):
                g_wait(h, p, i)
                s_start(h, p, i)

        if tail:
            t = n_full % D
            off = base + n_full * CH
            for h in (0, 1):
                s_wait(h, t, n_full - D)
                src_sl = table_h.at[idxs[h].at[pl.ds(n_full * CH, tail)]]
                dst_sl = rows[h][t].at[pl.ds(0, tail)]
                pltpu.async_copy(src_sl, dst_sl, gsem[h][t]).wait()
                pltpu.async_copy(dst_sl, outs[h].at[pl.ds(off, tail)],
                                 ssem[h][t]).wait()

        for i in range(n_full - D, n_full):
            p = i % D
            if tail and p == n_full % D:
                continue
            for h in (0, 1):
                s_wait(h, p, i)

    sems = [pltpu.SemaphoreType.DMA] * (4 * D + 1)
    return pl.kernel(
        body,
        out_type=(jax.ShapeDtypeStruct((E, WD), jnp.float32),
                  jax.ShapeDtypeStruct((E, WD), jnp.float32)),
        mesh=mesh,
        scratch_types=(
            [pltpu.VMEM((per_w,), jnp.int32)] * 2
            + [pltpu.VMEM((CH, WD), jnp.float32)] * (2 * D)
            + sems
        ),
        compiler_params=pltpu.CompilerParams(use_tc_tiling_on_sc=False),
        name="egnn_sc_gather",
    )


@functools.lru_cache(maxsize=None)
def _make_scatter(N, E):
    per_w = E // NW
    n_full = per_w // CH
    tail = per_w - n_full * CH
    rows_t = N // NS
    mesh = plsc.VectorSubcoreMesh(core_axis_name="c", subcore_axis_name="s")

    D = 4
    LA = 2
    n_main = (n_full // D) * D
    fired_max = n_main - 1 + LA

    def body(vals_h, dst_h, zeros_h, out_h, *scr):
        c = lax.axis_index("c")
        s = lax.axis_index("s")
        wid = s * NC + c
        base = wid * per_w

        idxb = scr[0:D]
        idxt = scr[D]
        valb = scr[D + 1:2 * D + 1]
        acc = scr[2 * D + 1]
        six = scr[2 * D + 2:3 * D + 2]
        sv = scr[3 * D + 2:4 * D + 2]
        sa = scr[4 * D + 2:5 * D + 2]

        pltpu.sync_copy(zeros_h, acc.at[pl.ds(s * rows_t, rows_t)])
        plsc.subcore_barrier()

        def ld_start(p, i):
            pltpu.async_copy(dst_h.at[pl.ds(base + i * CH, CH)], idxb[p],
                             six[p])
            pltpu.async_copy(vals_h.at[pl.ds(base + i * CH, CH)], valb[p],
                             sv[p])

        def ld_wait(p, i):
            pltpu.make_async_copy(dst_h.at[pl.ds(base + i * CH, CH)],
                                  idxb[p], six[p]).wait()
            pltpu.make_async_copy(vals_h.at[pl.ds(base + i * CH, CH)],
                                  valb[p], sv[p]).wait()

        def a_start(p):
            pltpu.async_copy(valb[p], acc.at[idxb[p]], sa[p], add=True)

        def a_wait(p):
            pltpu.make_async_copy(valb[p], acc.at[idxb[p]], sa[p]).wait()

        for k in range(LA):
            ld_start(k % D, k)

        def step_grp(g, carry):
            for p in range(D):
                i = g * D + p
                q = (p + LA) % D

                @pl.when(i + LA < n_full)
                def _(p=p, q=q, i=i):
                    @pl.when(i + LA >= D)
                    def _():
                        a_wait(q)
                    ld_start(q, i + LA)

                ld_wait(p, i)
                a_start(p)
            return carry
        lax.fori_loop(0, n_main // D, step_grp, 0)

        for i in range(n_main, n_full):
            p = i % D
            if i > fired_max:
                a_wait(p)
                ld_start(p, i)
            ld_wait(p, i)
            a_start(p)

        if tail:
            tp = n_full % D
            off = base + n_full * CH
            a_wait(tp)
            pltpu.sync_copy(dst_h.at[pl.ds(off, tail)], idxt)
            pltpu.sync_copy(vals_h.at[pl.ds(off, tail)],
                            valb[tp].at[pl.ds(0, tail)])
            pltpu.sync_copy(valb[tp].at[pl.ds(0, tail)], acc.at[idxt],
                            add=True)

        for i in range(n_full - D, n_full):
            p = i % D
            if tail and p == n_full % D:
                continue
            a_wait(p)

        plsc.subcore_barrier()
        pltpu.sync_copy(acc.at[pl.ds(s * rows_t, rows_t)],
                        out_h.at[pl.ds(c * N + s * rows_t, rows_t)])

    return pl.kernel(
        body,
        out_type=jax.ShapeDtypeStruct((NC * N, WD), jnp.float32),
        mesh=mesh,
        scratch_types=(
            [pltpu.VMEM((CH,), jnp.int32)] * D
            + [pltpu.VMEM((max(tail, 8),), jnp.int32)]
            + [pltpu.VMEM((CH, WD), jnp.float32)] * D
            + [pltpu.VMEM_SHARED((N, WD), jnp.float32)]
            + [pltpu.SemaphoreType.DMA] * (3 * D)
        ),
        compiler_params=pltpu.CompilerParams(use_tc_tiling_on_sc=False),
        name="egnn_sc_scatter",
    )


def _embed_pack(x, pos, embed_W, embed_b):
    N, F = x.shape
    BN = 2000

    def body(x_ref, p_ref, w_ref, b_ref, o_ref):
        feats = jnp.dot(x_ref[...], w_ref[...],
                        preferred_element_type=jnp.float32) + b_ref[...]
        o_ref[...] = jnp.concatenate(
            [feats, p_ref[...],
             jnp.zeros((BN, WD - H - 3), jnp.float32)], axis=1)

    return pl.pallas_call(
        body,
        grid=(N // BN,),
        in_specs=[
            pl.BlockSpec((BN, F), lambda i: (i, 0)),
            pl.BlockSpec((BN, 3), lambda i: (i, 0)),
            pl.BlockSpec((F, H), lambda i: (0, 0)),
            pl.BlockSpec((1, H), lambda i: (0, 0)),
        ],
        out_specs=pl.BlockSpec((BN, WD), lambda i: (i, 0)),
        out_shape=jax.ShapeDtypeStruct((N, WD), jnp.float32),
        name="egnn_embed_pack",
    )(x, pos, embed_W, embed_b.reshape(1, H))


SL = 4
VW = SL * WD


def _bdiag(w, rstep, cstep, roff=0):
    out = jnp.zeros((SL * rstep, SL * cstep), jnp.float32)
    for j in range(SL):
        out = out.at[j * rstep + roff:j * rstep + roff + w.shape[0],
                     j * cstep:j * cstep + w.shape[1]].set(w)
    return out


def _edge_mlp(sview, dview, attrs, W1, b1, W2, b2, cW1, cb1, cW2, cb2):
    EV = sview.shape[0]
    BV = 1600
    EH = W1.shape[1]

    wd_blk = _bdiag(W1[:H], WD, EH)
    ws_blk = _bdiag(W1[H:2 * H], WD, EH)
    w2_blk = _bdiag(W2, EH, H)
    cw1_blk = _bdiag(cW1, H, 4 * H)
    cw2_blk = _bdiag(cW2, 4 * H, 1)
    b1t = jnp.tile(b1.reshape(1, EH), (1, SL))
    b2t = jnp.tile(b2.reshape(1, H), (1, SL))
    cb1t = jnp.tile(cb1.reshape(1, 4 * H), (1, SL))
    cb2t = jnp.tile(cb2.reshape(1, 1), (1, SL))
    w1d = jnp.tile(W1[2 * H:2 * H + 1], (1, SL))
    w1a = jnp.tile(W1[2 * H + 1:2 * H + 2], (1, SL))
    lane = list(range(VW))
    mapd = jnp.asarray([[WD * (k // EH) + H for k in range(SL * EH)]],
                       jnp.int32)
    mapa = jnp.asarray([[k // EH for k in range(SL * EH)]], jnp.int32)
    mapm = jnp.asarray([[(l // WD) * H + (l % WD) % H for l in lane]],
                       jnp.int32)
    mapc = jnp.asarray([[l // WD for l in lane]], jnp.int32)
    maskm = jnp.asarray([[1.0 if (l % WD) < H else 0.0 for l in lane]],
                        jnp.float32)
    maskc = jnp.asarray([[1.0 if H <= (l % WD) < H + 3 else 0.0
                          for l in lane]], jnp.float32)

    def body(s_ref, d_ref, a0_ref, a1_ref, a2_ref, a3_ref,
             wd_ref, ws_ref, w2_ref, cw1_ref, cw2_ref,
             b1_ref, b2_ref, cb1_ref, cb2_ref, w1d_ref, w1a_ref,
             mapd_ref, mapa_ref, mapm_ref, mapc_ref,
             maskm_ref, maskc_ref, o_ref):
        sb = s_ref[...]
        db = d_ref[...]

        def b16(v):
            return v.astype(jnp.bfloat16).astype(jnp.float32)

        relall = sb - db
        relsq = relall * relall
        distf = (relsq + pltpu.roll(relsq, VW - 1, 1)
                 + pltpu.roll(relsq, VW - 2, 1))
        attr4 = jnp.concatenate(
            [a0_ref[...], a1_ref[...], a2_ref[...], a3_ref[...]], axis=1)

        dist_b = jnp.take_along_axis(
            distf, jnp.broadcast_to(mapd_ref[...], (BV, SL * EH)), axis=1)
        attr_b = jnp.take_along_axis(
            attr4, jnp.broadcast_to(mapa_ref[...], (BV, SL * EH)), axis=1)

        h = (jnp.dot(db, wd_ref[...], preferred_element_type=jnp.float32)
             + jnp.dot(sb, ws_ref[...], preferred_element_type=jnp.float32)
             + b16(dist_b) * b16(w1d_ref[...])
             + b16(attr_b) * b16(w1a_ref[...])
             + b1_ref[...])
        h1 = _silu(h)
        m_all = _silu(jnp.dot(h1, w2_ref[...],
                              preferred_element_type=jnp.float32) + b2_ref[...])
        chid = _silu(jnp.dot(m_all, cw1_ref[...],
                             preferred_element_type=jnp.float32) + cb1_ref[...])
        cw_all = jnp.dot(chid, cw2_ref[...],
                         preferred_element_type=jnp.float32) + cb2_ref[...]

        m_big = jnp.take_along_axis(
            m_all, jnp.broadcast_to(mapm_ref[...], (BV, VW)), axis=1)
        cw_big = jnp.take_along_axis(
            cw_all, jnp.broadcast_to(mapc_ref[...], (BV, VW)), axis=1)
        o_ref[...] = (m_big * maskm_ref[...]
                      + cw_big * relall * maskc_ref[...])

    full = lambda i: (0, 0)
    blk = lambda i: (i, 0)
    return pl.pallas_call(
        body,
        grid=(EV // BV,),
        in_specs=[
            pl.BlockSpec((BV, VW), blk),
            pl.BlockSpec((BV, VW), blk),
            pl.BlockSpec((BV, 1), blk),
            pl.BlockSpec((BV, 1), blk),
            pl.BlockSpec((BV, 1), blk),
            pl.BlockSpec((BV, 1), blk),
            pl.BlockSpec(wd_blk.shape, full),
            pl.BlockSpec(ws_blk.shape, full),
            pl.BlockSpec(w2_blk.shape, full),
            pl.BlockSpec(cw1_blk.shape, full),
            pl.BlockSpec(cw2_blk.shape, full),
            pl.BlockSpec(b1t.shape, full),
            pl.BlockSpec(b2t.shape, full),
            pl.BlockSpec(cb1t.shape, full),
            pl.BlockSpec(cb2t.shape, full),
            pl.BlockSpec(w1d.shape, full),
            pl.BlockSpec(w1a.shape, full),
            pl.BlockSpec(mapd.shape, full),
            pl.BlockSpec(mapa.shape, full),
            pl.BlockSpec(mapm.shape, full),
            pl.BlockSpec(mapc.shape, full),
            pl.BlockSpec(maskm.shape, full),
            pl.BlockSpec(maskc.shape, full),
        ],
        out_specs=pl.BlockSpec((BV, VW), blk),
        out_shape=jax.ShapeDtypeStruct((EV, VW), jnp.float32),
        name="egnn_edge_mlp",
    )(sview, dview, attrs[0], attrs[1], attrs[2], attrs[3],
      wd_blk, ws_blk, w2_blk, cw1_blk, cw2_blk,
      b1t, b2t, cb1t, cb2t, w1d, w1a,
      mapd, mapa, mapm, mapc, maskm, maskc)


def _node_mlp(tview, pview, nW1, nb1, nW2, nb2):
    NV = tview.shape[0]
    BV = NV
    NH = nW1.shape[1]

    tw_blk = _bdiag(nW1[:H], WD, NH)
    pw_blk = _bdiag(nW1[H:2 * H], WD, NH)
    w2_blk = _bdiag(nW2, NH, H)
    b1t = jnp.tile(nb1.reshape(1, NH), (1, SL))
    b2t = jnp.tile(nb2.reshape(1, H), (1, SL))
    nb_blocks = NV // BV

    def body(t_ref, p_ref, tw_ref, pw_ref, w2_ref,
             b1_ref, b2_ref, o_ref):
        tb = t_ref[...]
        pb = p_ref[0] + p_ref[1]
        hmid = _silu(jnp.dot(tb, tw_ref[...],
                             preferred_element_type=jnp.float32)
                     + jnp.dot(pb, pw_ref[...],
                               preferred_element_type=jnp.float32)
                     + b1_ref[...])
        fdel = jnp.dot(hmid, w2_ref[...],
                       preferred_element_type=jnp.float32) + b2_ref[...]
        pieces = []
        for j in range(SL):
            c0 = j * WD
            pieces.append(tb[:, c0:c0 + H] + fdel[:, j * H:(j + 1) * H])
            pieces.append(tb[:, c0 + H:c0 + H + 3] + pb[:, c0 + H:c0 + H + 3])
            pieces.append(jnp.zeros((BV, WD - H - 3), jnp.float32))
        o_ref[...] = jnp.concatenate(pieces, axis=1)

    full = lambda i: (0, 0)
    blk = lambda i: (i, 0)
    return pl.pallas_call(
        body,
        grid=(nb_blocks,),
        in_specs=[
            pl.BlockSpec((BV, VW), blk),
            pl.BlockSpec((2, BV, VW), lambda i: (0, i, 0)),
            pl.BlockSpec(tw_blk.shape, full),
            pl.BlockSpec(pw_blk.shape, full),
            pl.BlockSpec(w2_blk.shape, full),
            pl.BlockSpec(b1t.shape, full),
            pl.BlockSpec(b2t.shape, full),
        ],
        out_specs=pl.BlockSpec((BV, VW), blk),
        out_shape=jax.ShapeDtypeStruct((NV, VW), jnp.float32),
        name="egnn_node_mlp",
    )(tview, pview.reshape(2, NV, VW), tw_blk, pw_blk, w2_blk, b1t, b2t)


def _final_lin(tview, lin_W, lin_b):
    NV = tview.shape[0]
    BV = NV
    C = lin_W.shape[1]

    lin_blk = _bdiag(lin_W, WD, C)
    bt = jnp.tile(lin_b.reshape(1, C), (1, SL))

    def body(t_ref, w_ref, b_ref, o_ref):
        o_ref[...] = jnp.dot(t_ref[...], w_ref[...],
                             preferred_element_type=jnp.float32) + b_ref[...]

    return pl.pallas_call(
        body,
        grid=(NV // BV,),
        in_specs=[
            pl.BlockSpec((BV, VW), lambda i: (i, 0)),
            pl.BlockSpec(lin_blk.shape, lambda i: (0, 0)),
            pl.BlockSpec(bt.shape, lambda i: (0, 0)),
        ],
        out_specs=pl.BlockSpec((BV, SL * C), lambda i: (i, 0)),
        out_shape=jax.ShapeDtypeStruct((NV, SL * C), jnp.float32),
        name="egnn_final_lin",
    )(tview, lin_blk, bt)


def kernel(x, edge_index, edge_attr, pos, embed_W, embed_b,
           edge_W1, edge_b1, edge_W2, edge_b2,
           coors_W1, coors_b1, coors_W2, coors_b2,
           node_W1, node_b1, node_W2, node_b2, lin_W, lin_b):
    N = x.shape[0]
    E = edge_index.shape[1]
    L = edge_W1.shape[0]
    assert E % NW == 0 and N % NS == 0

    src = edge_index[0]
    dst = edge_index[1]
    zeros_h = jnp.zeros((N // NS, WD), jnp.float32)
    attrs = [edge_attr[j::SL] for j in range(SL)]

    gather = _make_gather(N, E)
    scatter = _make_scatter(N, E)

    table = _embed_pack(x, pos, embed_W, embed_b)
    for l in range(L):
        srows, drows = gather(table, src, dst)
        evals_v = _edge_mlp(srows.reshape(E // SL, VW),
                            drows.reshape(E // SL, VW), attrs,
                            edge_W1[l], edge_b1[l], edge_W2[l], edge_b2[l],
                            coors_W1[l], coors_b1[l], coors_W2[l], coors_b2[l])
        parts = scatter(evals_v.reshape(E, WD), dst, zeros_h)
        tview = _node_mlp(table.reshape(N // SL, VW),
                          parts.reshape(2 * N // SL, VW),
                          node_W1[l], node_b1[l], node_W2[l], node_b2[l])
        table = tview.reshape(N, WD)
    out_v = _final_lin(table.reshape(N // SL, VW), lin_W, lin_b)
    return out_v.reshape(N, lin_W.shape[1])


# speedup vs baseline: 11.0896x; 1.0328x over previous
import functools

import jax
import jax.numpy as jnp
from jax import lax
from jax.experimental import pallas as pl
from jax.experimental.pallas import tpu as pltpu
from jax.experimental.pallas import tpu_sc as plsc

NC = 2
NS = 16
NW = NC * NS
CH = 128

WD = 32
H = 16


def _silu(v):
    return v * jax.nn.sigmoid(v)


@functools.lru_cache(maxsize=None)
def _make_gather(N, E):
    per_w = E // NW
    n_full = per_w // CH
    tail = per_w - n_full * CH
    mesh = plsc.VectorSubcoreMesh(core_axis_name="c", subcore_axis_name="s")

    D = 6
    K = 3
    n_main = (n_full // D) * D

    def body(table_h, src_h, dst_h, srows_h, drows_h, *scr):
        c = lax.axis_index("c")
        s = lax.axis_index("s")
        wid = s * NC + c
        base = wid * per_w

        idxs = scr[0:2]
        rows = (scr[2:2 + D], scr[2 + D:2 + 2 * D])
        gsem = (scr[2 + 2 * D:2 + 3 * D], scr[2 + 3 * D:2 + 4 * D])
        ssem = (scr[2 + 4 * D:2 + 5 * D], scr[2 + 5 * D:2 + 6 * D])
        sem_i = scr[2 + 6 * D]
        idxs_v, idxd_v = idxs
        outs = (srows_h, drows_h)

        pltpu.async_copy(src_h.at[pl.ds(base, per_w)], idxs_v, sem_i).wait()
        pltpu.async_copy(dst_h.at[pl.ds(base, per_w)], idxd_v, sem_i).wait()

        def g_start(h, p, i):
            return pltpu.async_copy(
                table_h.at[idxs[h].at[pl.ds(i * CH, CH)]], rows[h][p],
                gsem[h][p])

        def g_wait(h, p, i):
            pltpu.make_async_copy(
                table_h.at[idxs[h].at[pl.ds(i * CH, CH)]], rows[h][p],
                gsem[h][p]).wait()

        def s_start(h, p, i):
            return pltpu.async_copy(
                rows[h][p], outs[h].at[pl.ds(base + i * CH, CH)], ssem[h][p])

        def s_wait(h, p, i):
            pltpu.make_async_copy(
                rows[h][p], outs[h].at[pl.ds(base + i * CH, CH)],
                ssem[h][p]).wait()

        for h in (0, 1):
            for p in range(K):
                g_start(h, p, p)

        def step_grp(g, carry):
            for p in range(D):
                i = g * D + p
                for h in (0, 1):
                    pf = (p + K) % D

                    @pl.when(i + K < n_full)
                    def _(h=h, pf=pf, i=i):
                        @pl.when(i + K >= D)
                        def _():
                            s_wait(h, pf, i + K - D)
                        g_start(h, pf, i + K)

                    g_wait(h, p, i)
                    s_start(h, p, i)
            return carry
        lax.fori_loop(0, n_main // D, step_grp, 0)

        assert n_full - n_main <= K
        for i in range(n_main, n_full):
            p = i % D
            for h in (0, 1):
                g_wait(h, p, i)
                s_start(h, p, i)

        if tail:
            t = n_full % D
            off = base + n_full * CH
            for h in (0, 1):
                s_wait(h, t, n_full - D)
                src_sl = table_h.at[idxs[h].at[pl.ds(n_full * CH, tail)]]
                dst_sl = rows[h][t].at[pl.ds(0, tail)]
                pltpu.async_copy(src_sl, dst_sl, gsem[h][t]).wait()
                pltpu.async_copy(dst_sl, outs[h].at[pl.ds(off, tail)],
                                 ssem[h][t]).wait()

        for i in range(n_full - D, n_full):
            p = i % D
            if tail and p == n_full % D:
                continue
            for h in (0, 1):
                s_wait(h, p, i)

    sems = [pltpu.SemaphoreType.DMA] * (4 * D + 1)
    return pl.kernel(
        body,
        out_type=(jax.ShapeDtypeStruct((E, WD), jnp.float32),
                  jax.ShapeDtypeStruct((E, WD), jnp.float32)),
        mesh=mesh,
        scratch_types=(
            [pltpu.VMEM((per_w,), jnp.int32)] * 2
            + [pltpu.VMEM((CH, WD), jnp.float32)] * (2 * D)
            + sems
        ),
        compiler_params=pltpu.CompilerParams(use_tc_tiling_on_sc=False),
        name="egnn_sc_gather",
    )


@functools.lru_cache(maxsize=None)
def _make_scatter(N, E):
    per_w = E // NW
    n_full = per_w // CH
    tail = per_w - n_full * CH
    rows_t = N // NS
    mesh = plsc.VectorSubcoreMesh(core_axis_name="c", subcore_axis_name="s")

    D = 4
    LA = 2
    n_main = (n_full // D) * D
    fired_max = n_main - 1 + LA

    def body(vals_h, dst_h, zeros_h, out_h, *scr):
        c = lax.axis_index("c")
        s = lax.axis_index("s")
        wid = s * NC + c
        base = wid * per_w

        idxb = scr[0:D]
        idxt = scr[D]
        valb = scr[D + 1:2 * D + 1]
        acc = scr[2 * D + 1]
        six = scr[2 * D + 2:3 * D + 2]
        sv = scr[3 * D + 2:4 * D + 2]
        sa = scr[4 * D + 2:5 * D + 2]

        pltpu.sync_copy(zeros_h, acc.at[pl.ds(s * rows_t, rows_t)])
        plsc.subcore_barrier()

        def ld_start(p, i):
            pltpu.async_copy(dst_h.at[pl.ds(base + i * CH, CH)], idxb[p],
                             six[p])
            pltpu.async_copy(vals_h.at[pl.ds(base + i * CH, CH)], valb[p],
                             sv[p])

        def ld_wait(p, i):
            pltpu.make_async_copy(dst_h.at[pl.ds(base + i * CH, CH)],
                                  idxb[p], six[p]).wait()
            pltpu.make_async_copy(vals_h.at[pl.ds(base + i * CH, CH)],
                                  valb[p], sv[p]).wait()

        def a_start(p):
            pltpu.async_copy(valb[p], acc.at[idxb[p]], sa[p], add=True)

        def a_wait(p):
            pltpu.make_async_copy(valb[p], acc.at[idxb[p]], sa[p]).wait()

        for k in range(LA):
            ld_start(k % D, k)

        def step_grp(g, carry):
            for p in range(D):
                i = g * D + p
                q = (p + LA) % D

                @pl.when(i + LA < n_full)
                def _(p=p, q=q, i=i):
                    @pl.when(i + LA >= D)
                    def _():
                        a_wait(q)
                    ld_start(q, i + LA)

                ld_wait(p, i)
                a_start(p)
            return carry
        lax.fori_loop(0, n_main // D, step_grp, 0)

        for i in range(n_main, n_full):
            p = i % D
            if i > fired_max:
                a_wait(p)
                ld_start(p, i)
            ld_wait(p, i)
            a_start(p)

        if tail:
            tp = n_full % D
            off = base + n_full * CH
            a_wait(tp)
            pltpu.sync_copy(dst_h.at[pl.ds(off, tail)], idxt)
            pltpu.sync_copy(vals_h.at[pl.ds(off, tail)],
                            valb[tp].at[pl.ds(0, tail)])
            pltpu.sync_copy(valb[tp].at[pl.ds(0, tail)], acc.at[idxt],
                            add=True)

        for i in range(n_full - D, n_full):
            p = i % D
            if tail and p == n_full % D:
                continue
            a_wait(p)

        plsc.subcore_barrier()
        pltpu.sync_copy(acc.at[pl.ds(s * rows_t, rows_t)],
                        out_h.at[pl.ds(c * N + s * rows_t, rows_t)])

    return pl.kernel(
        body,
        out_type=jax.ShapeDtypeStruct((NC * N, WD), jnp.float32),
        mesh=mesh,
        scratch_types=(
            [pltpu.VMEM((CH,), jnp.int32)] * D
            + [pltpu.VMEM((max(tail, 8),), jnp.int32)]
            + [pltpu.VMEM((CH, WD), jnp.float32)] * D
            + [pltpu.VMEM_SHARED((N, WD), jnp.float32)]
            + [pltpu.SemaphoreType.DMA] * (3 * D)
        ),
        compiler_params=pltpu.CompilerParams(use_tc_tiling_on_sc=False),
        name="egnn_sc_scatter",
    )


def _embed_pack(x, pos, embed_W, embed_b):
    N, F = x.shape
    BN = 2000

    def body(x_ref, p_ref, w_ref, b_ref, o_ref):
        feats = jnp.dot(x_ref[...], w_ref[...],
                        preferred_element_type=jnp.float32) + b_ref[...]
        o_ref[...] = jnp.concatenate(
            [feats, p_ref[...],
             jnp.zeros((BN, WD - H - 3), jnp.float32)], axis=1)

    return pl.pallas_call(
        body,
        grid=(N // BN,),
        in_specs=[
            pl.BlockSpec((BN, F), lambda i: (i, 0)),
            pl.BlockSpec((BN, 3), lambda i: (i, 0)),
            pl.BlockSpec((F, H), lambda i: (0, 0)),
            pl.BlockSpec((1, H), lambda i: (0, 0)),
        ],
        out_specs=pl.BlockSpec((BN, WD), lambda i: (i, 0)),
        out_shape=jax.ShapeDtypeStruct((N, WD), jnp.float32),
        name="egnn_embed_pack",
    )(x, pos, embed_W, embed_b.reshape(1, H))


SL = 4
VW = SL * WD


def _bdiag(w, rstep, cstep, roff=0):
    out = jnp.zeros((SL * rstep, SL * cstep), jnp.float32)
    for j in range(SL):
        out = out.at[j * rstep + roff:j * rstep + roff + w.shape[0],
                     j * cstep:j * cstep + w.shape[1]].set(w)
    return out


def _edge_mlp(sview, dview, attrs, W1, b1, W2, b2, cW1, cb1, cW2, cb2):
    EV = sview.shape[0]
    BV = next(bv for bv in range(min(EV, 4200) // 8 * 8, 0, -8)
              if EV % bv == 0)
    EH = W1.shape[1]

    wd_blk = _bdiag(W1[:H], WD, EH)
    ws_blk = _bdiag(W1[H:2 * H], WD, EH)
    w2_blk = _bdiag(W2, EH, H)
    cw1_blk = _bdiag(cW1, H, 4 * H)
    cw2_blk = _bdiag(cW2, 4 * H, 1)
    b1t = jnp.tile(b1.reshape(1, EH), (1, SL))
    b2t = jnp.tile(b2.reshape(1, H), (1, SL))
    cb1t = jnp.tile(cb1.reshape(1, 4 * H), (1, SL))
    cb2t = jnp.tile(cb2.reshape(1, 1), (1, SL))
    w1d = jnp.tile(W1[2 * H:2 * H + 1], (1, SL))
    w1a = jnp.tile(W1[2 * H + 1:2 * H + 2], (1, SL))
    lane = list(range(VW))
    mapd = jnp.asarray([[WD * (k // EH) + H for k in range(SL * EH)]],
                       jnp.int32)
    mapa = jnp.asarray([[k // EH for k in range(SL * EH)]], jnp.int32)
    mapm = jnp.asarray([[(l // WD) * H + (l % WD) % H for l in lane]],
                       jnp.int32)
    mapc = jnp.asarray([[l // WD for l in lane]], jnp.int32)
    maskm = jnp.asarray([[1.0 if (l % WD) < H else 0.0 for l in lane]],
                        jnp.float32)
    maskc = jnp.asarray([[1.0 if H <= (l % WD) < H + 3 else 0.0
                          for l in lane]], jnp.float32)

    def body(s_ref, d_ref, a0_ref, a1_ref, a2_ref, a3_ref,
             wd_ref, ws_ref, w2_ref, cw1_ref, cw2_ref,
             b1_ref, b2_ref, cb1_ref, cb2_ref, w1d_ref, w1a_ref,
             mapd_ref, mapa_ref, mapm_ref, mapc_ref,
             maskm_ref, maskc_ref, o_ref):
        sb = s_ref[...]
        db = d_ref[...]

        def b16(v):
            return v.astype(jnp.bfloat16).astype(jnp.float32)

        relall = sb - db
        relsq = relall * relall
        distf = (relsq + pltpu.roll(relsq, VW - 1, 1)
                 + pltpu.roll(relsq, VW - 2, 1))
        attr4 = jnp.concatenate(
            [a0_ref[...], a1_ref[...], a2_ref[...], a3_ref[...]], axis=1)

        dist_b = jnp.take_along_axis(
            distf, jnp.broadcast_to(mapd_ref[...], (BV, SL * EH)), axis=1)
        attr_b = jnp.take_along_axis(
            attr4, jnp.broadcast_to(mapa_ref[...], (BV, SL * EH)), axis=1)

        h = (jnp.dot(db, wd_ref[...], preferred_element_type=jnp.float32)
             + jnp.dot(sb, ws_ref[...], preferred_element_type=jnp.float32)
             + b16(dist_b) * b16(w1d_ref[...])
             + b16(attr_b) * b16(w1a_ref[...])
             + b1_ref[...])
        h1 = _silu(h)
        m_all = _silu(jnp.dot(h1, w2_ref[...],
                              preferred_element_type=jnp.float32) + b2_ref[...])
        chid = _silu(jnp.dot(m_all, cw1_ref[...],
                             preferred_element_type=jnp.float32) + cb1_ref[...])
        cw_all = jnp.dot(chid, cw2_ref[...],
                         preferred_element_type=jnp.float32) + cb2_ref[...]

        m_big = jnp.take_along_axis(
            m_all, jnp.broadcast_to(mapm_ref[...], (BV, VW)), axis=1)
        cw_big = jnp.take_along_axis(
            cw_all, jnp.broadcast_to(mapc_ref[...], (BV, VW)), axis=1)
        o_ref[...] = (m_big * maskm_ref[...]
                      + cw_big * relall * maskc_ref[...])

    full = lambda i: (0, 0)
    blk = lambda i: (i, 0)
    return pl.pallas_call(
        body,
        grid=(EV // BV,),
        in_specs=[
            pl.BlockSpec((BV, VW), blk),
            pl.BlockSpec((BV, VW), blk),
            pl.BlockSpec((BV, 1), blk),
            pl.BlockSpec((BV, 1), blk),
            pl.BlockSpec((BV, 1), blk),
            pl.BlockSpec((BV, 1), blk),
            pl.BlockSpec(wd_blk.shape, full),
            pl.BlockSpec(ws_blk.shape, full),
            pl.BlockSpec(w2_blk.shape, full),
            pl.BlockSpec(cw1_blk.shape, full),
            pl.BlockSpec(cw2_blk.shape, full),
            pl.BlockSpec(b1t.shape, full),
            pl.BlockSpec(b2t.shape, full),
            pl.BlockSpec(cb1t.shape, full),
            pl.BlockSpec(cb2t.shape, full),
            pl.BlockSpec(w1d.shape, full),
            pl.BlockSpec(w1a.shape, full),
            pl.BlockSpec(mapd.shape, full),
            pl.BlockSpec(mapa.shape, full),
            pl.BlockSpec(mapm.shape, full),
            pl.BlockSpec(mapc.shape, full),
            pl.BlockSpec(maskm.shape, full),
            pl.BlockSpec(maskc.shape, full),
        ],
        out_specs=pl.BlockSpec((BV, VW), blk),
        out_shape=jax.ShapeDtypeStruct((EV, VW), jnp.float32),
        name="egnn_edge_mlp",
    )(sview, dview, attrs[0], attrs[1], attrs[2], attrs[3],
      wd_blk, ws_blk, w2_blk, cw1_blk, cw2_blk,
      b1t, b2t, cb1t, cb2t, w1d, w1a,
      mapd, mapa, mapm, mapc, maskm, maskc)


def _node_mlp(tview, pview, pview2, nW1, nb1, nW2, nb2):
    NV = tview.shape[0]
    BV = NV
    NH = nW1.shape[1]

    tw_blk = _bdiag(nW1[:H], WD, NH)
    pw_blk = _bdiag(nW1[H:2 * H], WD, NH)
    w2_blk = _bdiag(nW2, NH, H)
    b1t = jnp.tile(nb1.reshape(1, NH), (1, SL))
    b2t = jnp.tile(nb2.reshape(1, H), (1, SL))
    nb_blocks = NV // BV

    def body(t_ref, p_ref, q_ref, tw_ref, pw_ref, w2_ref,
             b1_ref, b2_ref, o_ref):
        tb = t_ref[...]
        pb = (p_ref[0] + p_ref[1]) + (q_ref[0] + q_ref[1])
        hmid = _silu(jnp.dot(tb, tw_ref[...],
                             preferred_element_type=jnp.float32)
                     + jnp.dot(pb, pw_ref[...],
                               preferred_element_type=jnp.float32)
                     + b1_ref[...])
        fdel = jnp.dot(hmid, w2_ref[...],
                       preferred_element_type=jnp.float32) + b2_ref[...]
        pieces = []
        for j in range(SL):
            c0 = j * WD
            pieces.append(tb[:, c0:c0 + H] + fdel[:, j * H:(j + 1) * H])
            pieces.append(tb[:, c0 + H:c0 + H + 3] + pb[:, c0 + H:c0 + H + 3])
            pieces.append(jnp.zeros((BV, WD - H - 3), jnp.float32))
        o_ref[...] = jnp.concatenate(pieces, axis=1)

    full = lambda i: (0, 0)
    blk = lambda i: (i, 0)
    return pl.pallas_call(
        body,
        grid=(nb_blocks,),
        in_specs=[
            pl.BlockSpec((BV, VW), blk),
            pl.BlockSpec((2, BV, VW), lambda i: (0, i, 0)),
            pl.BlockSpec((2, BV, VW), lambda i: (0, i, 0)),
            pl.BlockSpec(tw_blk.shape, full),
            pl.BlockSpec(pw_blk.shape, full),
            pl.BlockSpec(w2_blk.shape, full),
            pl.BlockSpec(b1t.shape, full),
            pl.BlockSpec(b2t.shape, full),
        ],
        out_specs=pl.BlockSpec((BV, VW), blk),
        out_shape=jax.ShapeDtypeStruct((NV, VW), jnp.float32),
        name="egnn_node_mlp",
    )(tview, pview.reshape(2, NV, VW), pview2.reshape(2, NV, VW),
      tw_blk, pw_blk, w2_blk, b1t, b2t)


def _final_lin(tview, lin_W, lin_b):
    NV = tview.shape[0]
    BV = NV
    C = lin_W.shape[1]

    lin_blk = _bdiag(lin_W, WD, C)
    bt = jnp.tile(lin_b.reshape(1, C), (1, SL))

    def body(t_ref, w_ref, b_ref, o_ref):
        o_ref[...] = jnp.dot(t_ref[...], w_ref[...],
                             preferred_element_type=jnp.float32) + b_ref[...]

    return pl.pallas_call(
        body,
        grid=(NV // BV,),
        in_specs=[
            pl.BlockSpec((BV, VW), lambda i: (i, 0)),
            pl.BlockSpec(lin_blk.shape, lambda i: (0, 0)),
            pl.BlockSpec(bt.shape, lambda i: (0, 0)),
        ],
        out_specs=pl.BlockSpec((BV, SL * C), lambda i: (i, 0)),
        out_shape=jax.ShapeDtypeStruct((NV, SL * C), jnp.float32),
        name="egnn_final_lin",
    )(tview, lin_blk, bt)


def kernel(x, edge_index, edge_attr, pos, embed_W, embed_b,
           edge_W1, edge_b1, edge_W2, edge_b2,
           coors_W1, coors_b1, coors_W2, coors_b2,
           node_W1, node_b1, node_W2, node_b2, lin_W, lin_b):
    N = x.shape[0]
    E = edge_index.shape[1]
    L = edge_W1.shape[0]
    assert E % NW == 0 and N % NS == 0

    EhA = (E // 2 + 255) // 256 * 256
    EhB = E - EhA
    assert EhA % 256 == 0 and EhB % 256 == 0
    src = edge_index[0]
    dst = edge_index[1]
    zeros_h = jnp.zeros((N // NS, WD), jnp.float32)
    attrs = [edge_attr[j::SL] for j in range(SL)]
    halves = [
        (EhA, src[:EhA], dst[:EhA], [a[:EhA // SL] for a in attrs]),
        (EhB, src[EhA:], dst[EhA:], [a[EhA // SL:] for a in attrs]),
    ]

    table = _embed_pack(x, pos, embed_W, embed_b)
    for l in range(L):
        parts = []
        for Eh, hsrc, hdst, hattrs in halves:
            srows, drows = _make_gather(N, Eh)(table, hsrc, hdst)
            evals_v = _edge_mlp(srows.reshape(Eh // SL, VW),
                                drows.reshape(Eh // SL, VW), hattrs,
                                edge_W1[l], edge_b1[l], edge_W2[l],
                                edge_b2[l], coors_W1[l], coors_b1[l],
                                coors_W2[l], coors_b2[l])
            parts.append(_make_scatter(N, Eh)(evals_v.reshape(Eh, WD),
                                              hdst, zeros_h))
        tview = _node_mlp(table.reshape(N // SL, VW),
                          parts[0].reshape(2 * N // SL, VW),
                          parts[1].reshape(2 * N // SL, VW),
                          node_W1[l], node_b1[l], node_W2[l], node_b2[l])
        table = tview.reshape(N, WD)
    out_v = _final_lin(table.reshape(N // SL, VW), lin_W, lin_b)
    return out_v.reshape(N, lin_W.shape[1])
